# Initial kernel scaffold; baseline (speedup 1.0000x reference)
import jax
import jax.numpy as jnp
from jax.experimental import pallas as pl


def kernel(x, edge_index, batch, static_feature, W1l, b1l, W1r, W2l, b2l, W2r, Wfc1, bfc1, Wfc2, bfc2, Wp, bp):
    raise NotImplementedError("write your pallas kernel here")


# speedup vs baseline: 2.0980x; 2.0980x over previous
import functools

import jax
import jax.numpy as jnp
from jax import lax
from jax.experimental import pallas as pl
from jax.experimental.pallas import tpu as pltpu
from jax.experimental.pallas import tpu_sc as plsc

N = 10000
E = 160000
D = 256
H = 512
G = 64
SF = 4

NC = 2
NS = 16
NW = NC * NS
WH = 128
TB = 128
NB = 40
E2 = NW * NB * TB
NP = 10240
STRIPE = NP // NS
ZB = 128
ZCH = STRIPE // ZB
CH = 8
NQ = NB // CH


def _sc_agg_body(P, with_cnt, *refs):
    if with_cnt:
        (table, src_off, dst_idx, zrows, ones,
         out_sums, out_cnt,
         srcv, dstv, rows, sem, acc) = refs
    else:
        (table, src_off, dst_idx, zrows,
         out_sums,
         srcv, dstv, rows, sem, acc) = refs

    c = lax.axis_index("c")
    s = lax.axis_index("s")
    tid = s * NC + c
    lo = s * STRIPE

    def zero_stripe():
        for z in range(ZCH):
            pltpu.sync_copy(zrows, acc.at[pl.ds(lo + z * ZB, ZB)])

    def copy_stripe_out(dst_ref):
        for z in range(ZCH):
            pltpu.sync_copy(acc.at[pl.ds(lo + z * ZB, ZB)],
                            dst_ref.at[pl.ds(lo + z * ZB, ZB)])

    def edge_loop(gather):
        for q in range(NQ):
            pltpu.sync_copy(dst_idx.at[tid, pl.ds(q * CH, CH)], dstv)
            if gather is not None:
                pltpu.sync_copy(src_off.at[gather, tid, pl.ds(q * CH, CH)],
                                srcv)

            def body(b, carry):
                if gather is not None:
                    pltpu.async_copy(table.at[srcv.at[b]], rows, sem).wait()
                pltpu.sync_copy(rows, acc.at[dstv.at[b]], add=True)
                return carry

            lax.fori_loop(0, CH, body, 0)

    if with_cnt:
        pltpu.sync_copy(ones, rows)
        zero_stripe()
        plsc.subcore_barrier()
        edge_loop(None)
        plsc.subcore_barrier()
        copy_stripe_out(out_cnt.at[c])

    for p in range(P):
        zero_stripe()
        plsc.subcore_barrier()
        edge_loop(p)
        plsc.subcore_barrier()
        copy_stripe_out(out_sums.at[c, p])


def _make_sc_agg(P, with_cnt):
    out_type = [jax.ShapeDtypeStruct((NC, P, NP, WH), jnp.float32)]
    if with_cnt:
        out_type.append(jax.ShapeDtypeStruct((NC, NP, WH), jnp.float32))
    scratch = [
        pltpu.VMEM((CH, TB), jnp.int32),
        pltpu.VMEM((CH, TB), jnp.int32),
        pltpu.VMEM((TB, WH), jnp.float32),
        pltpu.SemaphoreType.DMA,
        pltpu.VMEM_SHARED((NP, WH), jnp.float32),
    ]
    mesh = plsc.VectorSubcoreMesh(core_axis_name="c", subcore_axis_name="s",
                                  num_cores=NC, num_subcores=NS)
    return pl.kernel(functools.partial(_sc_agg_body, P, with_cnt),
                     out_type=out_type, mesh=mesh, scratch_types=scratch)


def _tc_layer1(sums_ref, cnt_ref, x_ref, wl_ref, wr_ref, b_ref, h_ref):
    sm = sums_ref[...]
    agg = sm[0] + sm[1]
    aggr = jnp.concatenate([agg[0], agg[1]], axis=1)
    cnt = cnt_ref[0, :, 0] + cnt_ref[1, :, 0]
    aggr = aggr / jnp.maximum(cnt, 1.0)[:, None]
    out = (jnp.dot(aggr, wl_ref[...], preferred_element_type=jnp.float32)
           + b_ref[...]
           + jnp.dot(x_ref[...], wr_ref[...], preferred_element_type=jnp.float32))
    nrm = jnp.sqrt(jnp.sum(out * out, axis=1, keepdims=True))
    out = out / jnp.maximum(nrm, 1e-12)
    h_ref[...] = jnp.maximum(out, 0.0)


def _tc_layer2(sums_ref, cnt_ref, h_ref, wl_ref, wr_ref, b_ref, batch_ref,
               pooled_ref):
    i = pl.program_id(0)
    sm = sums_ref[...]
    agg = sm[0] + sm[1]
    aggr = jnp.concatenate([agg[0], agg[1], agg[2], agg[3]], axis=1)
    cnt = cnt_ref[0, :, 0] + cnt_ref[1, :, 0]
    aggr = aggr / jnp.maximum(cnt, 1.0)[:, None]
    out = (jnp.dot(aggr, wl_ref[...], preferred_element_type=jnp.float32)
           + b_ref[...]
           + jnp.dot(h_ref[...], wr_ref[...], preferred_element_type=jnp.float32))
    nrm = jnp.sqrt(jnp.sum(out * out, axis=1, keepdims=True))
    out = out / jnp.maximum(nrm, 1e-12)
    out = jnp.maximum(out, 0.0)

    bids = batch_ref[0, 0, :]
    gids = lax.broadcasted_iota(jnp.int32, (G, out.shape[0]), 0)
    mask = (gids == bids[None, :]).astype(jnp.float32)
    part = jnp.dot(mask, out, preferred_element_type=jnp.float32)

    @pl.when(i == 0)
    def _():
        pooled_ref[...] = jnp.zeros_like(pooled_ref)

    pooled_ref[...] += part


def _tc_head(pooled_ref, sfeat_ref, w1a_ref, w1b_ref, b1_ref, w2_ref, b2_ref,
             wp_ref, bp_ref, pred_ref):
    z = (jnp.dot(pooled_ref[...], w1a_ref[...], preferred_element_type=jnp.float32)
         + jnp.dot(sfeat_ref[...], w1b_ref[...], preferred_element_type=jnp.float32)
         + b1_ref[...])
    z = jnp.maximum(z, 0.0)
    z = jnp.dot(z, w2_ref[...], preferred_element_type=jnp.float32) + b2_ref[...]
    z = jnp.maximum(z, 0.0)
    t = jnp.sum(z * wp_ref[...], axis=1, keepdims=True) + bp_ref[...]
    pred_ref[...] = jnp.maximum(-t, 0.0) + jnp.log1p(jnp.exp(-jnp.abs(t)))


def kernel(x, edge_index, batch, static_feature, W1l, b1l, W1r, W2l, b2l, W2r,
           Wfc1, bfc1, Wfc2, bfc2, Wp, bp):
    src = edge_index[0]
    dst = edge_index[1]
    f32 = jnp.float32

    pad = E2 - E
    srcp = jnp.concatenate([src, jnp.zeros((pad,), jnp.int32)])
    dstp = jnp.concatenate([dst, jnp.full((pad,), NP - 1, jnp.int32)])
    dst_idx = dstp.reshape(NW, NB, TB)
    offs2 = jnp.arange(2, dtype=jnp.int32)[:, None] * N
    offs4 = jnp.arange(4, dtype=jnp.int32)[:, None] * N
    src_off1 = (srcp[None, :] + offs2).reshape(2, NW, NB, TB)
    src_off2 = (srcp[None, :] + offs4).reshape(4, NW, NB, TB)
    x_table = x.reshape(N, 2, WH).transpose(1, 0, 2).reshape(2 * N, WH)
    zrows = jnp.zeros((ZB, WH), f32)
    ones = jnp.ones((TB, WH), f32)

    sums1, cnt = _make_sc_agg(2, True)(x_table, src_off1, dst_idx, zrows, ones)

    R = 400
    grid = (N // R,)
    h = pl.pallas_call(
        _tc_layer1,
        grid=grid,
        in_specs=[
            pl.BlockSpec((NC, 2, R, WH), lambda i: (0, 0, i, 0)),
            pl.BlockSpec((NC, R, WH), lambda i: (0, i, 0)),
            pl.BlockSpec((R, D), lambda i: (i, 0)),
            pl.BlockSpec((D, H), lambda i: (0, 0)),
            pl.BlockSpec((D, H), lambda i: (0, 0)),
            pl.BlockSpec((1, H), lambda i: (0, 0)),
        ],
        out_specs=pl.BlockSpec((R, H), lambda i: (i, 0)),
        out_shape=jax.ShapeDtypeStruct((N, H), f32),
    )(sums1, cnt, x, W1l.T, W1r.T, b1l.reshape(1, H))

    h_table = h.reshape(N, 4, WH).transpose(1, 0, 2).reshape(4 * N, WH)
    (sums2,) = _make_sc_agg(4, False)(h_table, src_off2, dst_idx, zrows)

    batch3d = batch.reshape(N // R, 1, R)
    pooled = pl.pallas_call(
        _tc_layer2,
        grid=grid,
        in_specs=[
            pl.BlockSpec((NC, 4, R, WH), lambda i: (0, 0, i, 0)),
            pl.BlockSpec((NC, R, WH), lambda i: (0, i, 0)),
            pl.BlockSpec((R, H), lambda i: (i, 0)),
            pl.BlockSpec((H, H), lambda i: (0, 0)),
            pl.BlockSpec((H, H), lambda i: (0, 0)),
            pl.BlockSpec((1, H), lambda i: (0, 0)),
            pl.BlockSpec((1, 1, R), lambda i: (i, 0, 0)),
        ],
        out_specs=pl.BlockSpec((G, H), lambda i: (0, 0)),
        out_shape=jax.ShapeDtypeStruct((G, H), f32),
    )(sums2, cnt, h, W2l.T, W2r.T, b2l.reshape(1, H), batch3d)

    pred = pl.pallas_call(
        _tc_head,
        out_shape=jax.ShapeDtypeStruct((G, 1), f32),
    )(pooled, static_feature, Wfc1[:, :H].T, Wfc1[:, H:].T, bfc1.reshape(1, H),
      Wfc2.T, bfc2.reshape(1, H), Wp, bp.reshape(1, 1))

    return pred


# speedup vs baseline: 2.4092x; 1.1483x over previous
import functools

import jax
import jax.numpy as jnp
from jax import lax
from jax.experimental import pallas as pl
from jax.experimental.pallas import tpu as pltpu
from jax.experimental.pallas import tpu_sc as plsc

N = 10000
E = 160000
D = 256
H = 512
G = 64
SF = 4

NC = 2
NS = 16
NW = NC * NS
WH = 128
TB = 128
NB = 40
E2 = NW * NB * TB
NP = 10240
STRIPE = NP // NS
ZB = 128
ZCH = STRIPE // ZB
CH = 8
NQ = NB // CH


def _sc_agg_body(P, with_cnt, *refs):
    if with_cnt:
        (table, src_off, dst_idx, zrows, ones,
         out_sums, out_cnt,
         srcv, dstv, rows, sems, acc) = refs
    else:
        (table, src_off, dst_idx, zrows,
         out_sums,
         srcv, dstv, rows, sems, acc) = refs

    c = lax.axis_index("c")
    s = lax.axis_index("s")
    tid = s * NC + c
    lo = s * STRIPE

    def zero_stripe():
        for z in range(ZCH):
            pltpu.sync_copy(zrows, acc.at[pl.ds(lo + z * ZB, ZB)])

    def copy_stripe_out(dst_ref):
        for z in range(ZCH):
            pltpu.sync_copy(acc.at[pl.ds(lo + z * ZB, ZB)],
                            dst_ref.at[pl.ds(lo + z * ZB, ZB)])

    def edge_loop(gather):
        pltpu.sync_copy(dst_idx.at[tid, pl.ds(0, CH)], dstv.at[0])
        cps = [None, None]
        if gather is not None:
            pltpu.sync_copy(src_off.at[gather, tid, pl.ds(0, CH)], srcv.at[0])
            cps[0] = pltpu.async_copy(table.at[srcv.at[0, 0]], rows.at[0],
                                      sems[0])
        for i in range(NB):
            q, b = divmod(i, CH)
            nq, nb = divmod(i + 1, CH)
            if nb == 0 and nq < NQ:
                pltpu.sync_copy(dst_idx.at[tid, pl.ds(nq * CH, CH)],
                                dstv.at[nq % 2])
                if gather is not None:
                    pltpu.sync_copy(src_off.at[gather, tid, pl.ds(nq * CH, CH)],
                                    srcv.at[nq % 2])
            if gather is not None:
                if i + 1 < NB:
                    cps[(i + 1) % 2] = pltpu.async_copy(
                        table.at[srcv.at[nq % 2, nb]], rows.at[(i + 1) % 2],
                        sems[(i + 1) % 2])
                cps[i % 2].wait()
                pltpu.sync_copy(rows.at[i % 2], acc.at[dstv.at[q % 2, b]],
                                add=True)
            else:
                pltpu.sync_copy(rows.at[0], acc.at[dstv.at[q % 2, b]],
                                add=True)

    if with_cnt:
        pltpu.sync_copy(ones, rows.at[0])
        zero_stripe()
        plsc.subcore_barrier()
        edge_loop(None)
        plsc.subcore_barrier()
        copy_stripe_out(out_cnt.at[c])

    for p in range(P):
        zero_stripe()
        plsc.subcore_barrier()
        edge_loop(p)
        plsc.subcore_barrier()
        copy_stripe_out(out_sums.at[c, p])


def _make_sc_agg(P, with_cnt):
    out_type = [jax.ShapeDtypeStruct((NC, P, NP, WH), jnp.float32)]
    if with_cnt:
        out_type.append(jax.ShapeDtypeStruct((NC, NP, WH), jnp.float32))
    scratch = [
        pltpu.VMEM((2, CH, TB), jnp.int32),
        pltpu.VMEM((2, CH, TB), jnp.int32),
        pltpu.VMEM((2, TB, WH), jnp.float32),
        [pltpu.SemaphoreType.DMA, pltpu.SemaphoreType.DMA],
        pltpu.VMEM_SHARED((NP, WH), jnp.float32),
    ]
    mesh = plsc.VectorSubcoreMesh(core_axis_name="c", subcore_axis_name="s",
                                  num_cores=NC, num_subcores=NS)
    return pl.kernel(functools.partial(_sc_agg_body, P, with_cnt),
                     out_type=out_type, mesh=mesh, scratch_types=scratch)


def _tc_layer1(sums_ref, cnt_ref, x_ref, wl_ref, wr_ref, b_ref, h_ref):
    sm = sums_ref[...]
    agg = sm[0] + sm[1]
    aggr = jnp.concatenate([agg[0], agg[1]], axis=1)
    cnt = cnt_ref[0, :, 0] + cnt_ref[1, :, 0]
    aggr = aggr / jnp.maximum(cnt, 1.0)[:, None]
    out = (jnp.dot(aggr, wl_ref[...], preferred_element_type=jnp.float32)
           + b_ref[...]
           + jnp.dot(x_ref[...], wr_ref[...], preferred_element_type=jnp.float32))
    nrm = jnp.sqrt(jnp.sum(out * out, axis=1, keepdims=True))
    out = out / jnp.maximum(nrm, 1e-12)
    h_ref[...] = jnp.maximum(out, 0.0)


def _tc_layer2(sums_ref, cnt_ref, h_ref, wl_ref, wr_ref, b_ref, batch_ref,
               pooled_ref):
    i = pl.program_id(0)
    sm = sums_ref[...]
    agg = sm[0] + sm[1]
    aggr = jnp.concatenate([agg[0], agg[1], agg[2], agg[3]], axis=1)
    cnt = cnt_ref[0, :, 0] + cnt_ref[1, :, 0]
    aggr = aggr / jnp.maximum(cnt, 1.0)[:, None]
    out = (jnp.dot(aggr, wl_ref[...], preferred_element_type=jnp.float32)
           + b_ref[...]
           + jnp.dot(h_ref[...], wr_ref[...], preferred_element_type=jnp.float32))
    nrm = jnp.sqrt(jnp.sum(out * out, axis=1, keepdims=True))
    out = out / jnp.maximum(nrm, 1e-12)
    out = jnp.maximum(out, 0.0)

    bids = batch_ref[0, 0, :]
    gids = lax.broadcasted_iota(jnp.int32, (G, out.shape[0]), 0)
    mask = (gids == bids[None, :]).astype(jnp.float32)
    part = jnp.dot(mask, out, preferred_element_type=jnp.float32)

    @pl.when(i == 0)
    def _():
        pooled_ref[...] = jnp.zeros_like(pooled_ref)

    pooled_ref[...] += part


def _tc_head(pooled_ref, sfeat_ref, w1a_ref, w1b_ref, b1_ref, w2_ref, b2_ref,
             wp_ref, bp_ref, pred_ref):
    z = (jnp.dot(pooled_ref[...], w1a_ref[...], preferred_element_type=jnp.float32)
         + jnp.dot(sfeat_ref[...], w1b_ref[...], preferred_element_type=jnp.float32)
         + b1_ref[...])
    z = jnp.maximum(z, 0.0)
    z = jnp.dot(z, w2_ref[...], preferred_element_type=jnp.float32) + b2_ref[...]
    z = jnp.maximum(z, 0.0)
    t = jnp.sum(z * wp_ref[...], axis=1, keepdims=True) + bp_ref[...]
    pred_ref[...] = jnp.maximum(-t, 0.0) + jnp.log1p(jnp.exp(-jnp.abs(t)))


def kernel(x, edge_index, batch, static_feature, W1l, b1l, W1r, W2l, b2l, W2r,
           Wfc1, bfc1, Wfc2, bfc2, Wp, bp):
    src = edge_index[0]
    dst = edge_index[1]
    f32 = jnp.float32

    pad = E2 - E
    srcp = jnp.concatenate([src, jnp.zeros((pad,), jnp.int32)])
    dstp = jnp.concatenate([dst, jnp.full((pad,), NP - 1, jnp.int32)])
    dst_idx = dstp.reshape(NW, NB, TB)
    offs2 = jnp.arange(2, dtype=jnp.int32)[:, None] * N
    offs4 = jnp.arange(4, dtype=jnp.int32)[:, None] * N
    src_off1 = (srcp[None, :] + offs2).reshape(2, NW, NB, TB)
    src_off2 = (srcp[None, :] + offs4).reshape(4, NW, NB, TB)
    x_table = x.reshape(N, 2, WH).transpose(1, 0, 2).reshape(2 * N, WH)
    zrows = jnp.zeros((ZB, WH), f32)
    ones = jnp.ones((TB, WH), f32)

    sums1, cnt = _make_sc_agg(2, True)(x_table, src_off1, dst_idx, zrows, ones)

    R = 400
    grid = (N // R,)
    h = pl.pallas_call(
        _tc_layer1,
        grid=grid,
        in_specs=[
            pl.BlockSpec((NC, 2, R, WH), lambda i: (0, 0, i, 0)),
            pl.BlockSpec((NC, R, WH), lambda i: (0, i, 0)),
            pl.BlockSpec((R, D), lambda i: (i, 0)),
            pl.BlockSpec((D, H), lambda i: (0, 0)),
            pl.BlockSpec((D, H), lambda i: (0, 0)),
            pl.BlockSpec((1, H), lambda i: (0, 0)),
        ],
        out_specs=pl.BlockSpec((R, H), lambda i: (i, 0)),
        out_shape=jax.ShapeDtypeStruct((N, H), f32),
    )(sums1, cnt, x, W1l.T, W1r.T, b1l.reshape(1, H))

    h_table = h.reshape(N, 4, WH).transpose(1, 0, 2).reshape(4 * N, WH)
    (sums2,) = _make_sc_agg(4, False)(h_table, src_off2, dst_idx, zrows)

    batch3d = batch.reshape(N // R, 1, R)
    pooled = pl.pallas_call(
        _tc_layer2,
        grid=grid,
        in_specs=[
            pl.BlockSpec((NC, 4, R, WH), lambda i: (0, 0, i, 0)),
            pl.BlockSpec((NC, R, WH), lambda i: (0, i, 0)),
            pl.BlockSpec((R, H), lambda i: (i, 0)),
            pl.BlockSpec((H, H), lambda i: (0, 0)),
            pl.BlockSpec((H, H), lambda i: (0, 0)),
            pl.BlockSpec((1, H), lambda i: (0, 0)),
            pl.BlockSpec((1, 1, R), lambda i: (i, 0, 0)),
        ],
        out_specs=pl.BlockSpec((G, H), lambda i: (0, 0)),
        out_shape=jax.ShapeDtypeStruct((G, H), f32),
    )(sums2, cnt, h, W2l.T, W2r.T, b2l.reshape(1, H), batch3d)

    pred = pl.pallas_call(
        _tc_head,
        out_shape=jax.ShapeDtypeStruct((G, 1), f32),
    )(pooled, static_feature, Wfc1[:, :H].T, Wfc1[:, H:].T, bfc1.reshape(1, H),
      Wfc2.T, bfc2.reshape(1, H), Wp, bp.reshape(1, 1))

    return pred
